# baseline (speedup 1.0000x reference)
import functools

import jax
import jax.numpy as jnp
from jax import lax
from jax.experimental import pallas as pl
from jax.experimental.pallas import tpu as pltpu
from jax.experimental.pallas import tpu_sc as plsc

_NC = 2
_NS = 16
_NW = _NC * _NS
_LANES = 16
_IDX_CHUNK = 100
_LIST_LEN = 112


def _gather_pool_kernel(batch, hist, dim):
    n_chunk = hist // _IDX_CHUNK
    b_per_w = batch // _NW
    vregs = dim // _LANES
    row_w = 2 * dim
    n_blk = hist // _LANES
    rem = hist - n_blk * _LANES
    mesh = plsc.VectorSubcoreMesh(core_axis_name="c", subcore_axis_name="s")

    @functools.partial(
        pl.kernel,
        mesh=mesh,
        out_type=jax.ShapeDtypeStruct((batch, dim), jnp.float32),
        scratch_types=[
            pltpu.VMEM((b_per_w, n_chunk * _LIST_LEN), jnp.int32),
            pltpu.VMEM((b_per_w, n_chunk, _LIST_LEN), jnp.int32),
            pltpu.VMEM((_LIST_LEN, row_w), jnp.float32),
            pltpu.VMEM((_LIST_LEN, row_w), jnp.float32),
            pltpu.VMEM((b_per_w, dim), jnp.float32),
            pltpu.SemaphoreType.DMA,
            pltpu.SemaphoreType.DMA,
        ],
        compiler_params=pltpu.CompilerParams(
            use_tc_tiling_on_sc=True, needs_layout_passes=False
        ),
    )
    def k(ids_hbm, emb_hbm, out_hbm, idx_v, gidx_v, rows0, rows1, out_v,
          sem0, sem1):
        wid = lax.axis_index("s") * _NC + lax.axis_index("c")
        base = wid * b_per_w

        pltpu.sync_copy(ids_hbm.at[pl.ds(base, b_per_w)], idx_v)

        def prep_body(s, _):
            for c in range(n_chunk):
                for o in range(0, _LIST_LEN, _LANES):
                    gidx_v[s, c, pl.ds(o, _LANES)] = (
                        idx_v[s, pl.ds(c * _LIST_LEN + o, _LANES)] >> 1
                    )
            return 0

        lax.fori_loop(0, b_per_w, prep_body, 0)

        def gather(s, c, buf, sem):
            pltpu.async_copy(emb_hbm.at[gidx_v.at[s].at[c]], buf, sem)

        def drain(buf, sem):
            pltpu.make_async_copy(
                emb_hbm.at[gidx_v.at[0].at[0]], buf, sem
            ).wait()

        lane_iota = lax.iota(jnp.int32, _LANES)
        cn_blk = _IDX_CHUNK // _LANES
        c_rem = _IDX_CHUNK - cn_blk * _LANES

        def accum_chunk(buf, s, c, carry):
            cbase = c * _LIST_LEN

            def row_step(j, pvec, r, carry2):
                pj = jnp.sum(jnp.where(lane_iota == r, pvec, 0))
                take_hi = pj != 0
                out = []
                for v in range(vregs):
                    lo = buf[j, pl.ds(v * _LANES, _LANES)]
                    hi = buf[j, pl.ds(dim + v * _LANES, _LANES)]
                    out.append(carry2[v] + jnp.where(take_hi, hi, lo))
                return tuple(out)

            def blk_body(b, carry2):
                pvec = idx_v[s, pl.ds(cbase + b * _LANES, _LANES)] & 1

                def r_body(r, c2):
                    return row_step(b * _LANES + r, pvec, r, c2)

                return lax.fori_loop(0, _LANES, r_body, carry2)

            acc = lax.fori_loop(0, cn_blk, blk_body, carry)
            if c_rem:
                tb = cn_blk * _LANES
                pvec = idx_v[s, pl.ds(cbase + tb, _LANES)] & 1

                def tail_body(r, c2):
                    return row_step(tb + r, pvec, r, c2)

                acc = lax.fori_loop(0, c_rem, tail_body, acc)
            return acc

        zero = tuple(jnp.zeros((_LANES,), jnp.float32) for _ in range(vregs))
        gather(0, 0, rows0, sem0)

        def step(s, _):
            gather(s, 1, rows1, sem1)
            drain(rows0, sem0)
            acc = accum_chunk(rows0, s, 0, zero)
            gather(jnp.minimum(s + 1, b_per_w - 1), 0, rows0, sem0)
            drain(rows1, sem1)
            acc = accum_chunk(rows1, s, 1, acc)
            for v in range(vregs):
                out_v[s, pl.ds(v * _LANES, _LANES)] = acc[v]
            return 0

        lax.fori_loop(0, b_per_w, step, 0)
        drain(rows0, sem0)
        pltpu.sync_copy(out_v, out_hbm.at[pl.ds(base, b_per_w)])

    return k


def _mlp_body(inv_hist, sums_ref, w1_ref, b1_ref, w2_ref, b2_ref, out_ref):
    pooled = sums_ref[...] * inv_hist
    h = jnp.tanh(
        jnp.dot(pooled, w1_ref[...], preferred_element_type=jnp.float32)
        + b1_ref[...]
    )
    out_ref[...] = (
        jnp.dot(h, w2_ref[...], preferred_element_type=jnp.float32) + b2_ref[...]
    )


def kernel(input_ids, embedding, W1, b1, W2, b2):
    batch, hist = input_ids.shape
    vocab, dim = embedding.shape

    n_chunk = hist // _IDX_CHUNK
    ids = jnp.pad(
        input_ids.astype(jnp.int32).reshape(batch, n_chunk, _IDX_CHUNK),
        ((0, 0), (0, 0), (0, _LIST_LEN - _IDX_CHUNK)),
    ).reshape(batch, n_chunk * _LIST_LEN)
    emb2 = embedding.reshape(vocab // 2, 2 * dim)
    sums = _gather_pool_kernel(batch, hist, dim)(ids, emb2)

    out = pl.pallas_call(
        functools.partial(_mlp_body, 1.0 / hist),
        out_shape=jax.ShapeDtypeStruct((batch, 1), jnp.float32),
    )(sums, W1, b1.reshape(1, -1), W2, b2.reshape(1, 1))
    return out[:, 0]


# speedup vs baseline: 1.0016x; 1.0016x over previous
import functools

import jax
import jax.numpy as jnp
from jax import lax
from jax.experimental import pallas as pl
from jax.experimental.pallas import tpu as pltpu
from jax.experimental.pallas import tpu_sc as plsc

_NC = 2
_NS = 16
_NW = _NC * _NS
_LANES = 16
_IDX_CHUNK = 100
_LIST_LEN = 112


def _gather_pool_kernel(batch, hist, dim):
    n_chunk = hist // _IDX_CHUNK
    b_per_w = batch // _NW
    vregs = dim // _LANES
    row_w = 2 * dim
    n_blk = hist // _LANES
    rem = hist - n_blk * _LANES
    mesh = plsc.VectorSubcoreMesh(core_axis_name="c", subcore_axis_name="s")

    @functools.partial(
        pl.kernel,
        mesh=mesh,
        out_type=jax.ShapeDtypeStruct((batch, dim), jnp.float32),
        scratch_types=[
            pltpu.VMEM((b_per_w, n_chunk * _LIST_LEN), jnp.int32),
            pltpu.VMEM((b_per_w, _LANES), jnp.int32),
            pltpu.VMEM((b_per_w, n_chunk, _LIST_LEN), jnp.int32),
            pltpu.VMEM((_LIST_LEN, row_w), jnp.float32),
            pltpu.VMEM((_LIST_LEN, row_w), jnp.float32),
            pltpu.VMEM((b_per_w, dim), jnp.float32),
            pltpu.SemaphoreType.DMA,
            pltpu.SemaphoreType.DMA,
        ],
        compiler_params=pltpu.CompilerParams(
            use_tc_tiling_on_sc=True, needs_layout_passes=False
        ),
    )
    def k(ids_hbm, cnt_hbm, emb_hbm, out_hbm, idx_v, cnt_v, gidx_v,
          rows0, rows1, out_v, sem0, sem1):
        wid = lax.axis_index("s") * _NC + lax.axis_index("c")
        base = wid * b_per_w

        pltpu.sync_copy(ids_hbm.at[pl.ds(base, b_per_w)], idx_v)
        pltpu.sync_copy(cnt_hbm.at[pl.ds(base, b_per_w)], cnt_v)

        def prep_body(s, _):
            for c in range(n_chunk):
                for o in range(0, _LIST_LEN, _LANES):
                    gidx_v[s, c, pl.ds(o, _LANES)] = (
                        idx_v[s, pl.ds(c * _LIST_LEN + o, _LANES)] >> 1
                    )
            return 0

        lax.fori_loop(0, b_per_w, prep_body, 0)

        def gather(s, c, buf, sem):
            pltpu.async_copy(emb_hbm.at[gidx_v.at[s].at[c]], buf, sem)

        def drain(buf, sem):
            pltpu.make_async_copy(
                emb_hbm.at[gidx_v.at[0].at[0]], buf, sem
            ).wait()

        def accum_chunk(buf, ec, carry):
            def lo_body(j, c2):
                return tuple(
                    c2[v] + buf[j, pl.ds(v * _LANES, _LANES)]
                    for v in range(vregs)
                )

            def hi_body(j, c2):
                return tuple(
                    c2[v] + buf[j, pl.ds(dim + v * _LANES, _LANES)]
                    for v in range(vregs)
                )

            acc = lax.fori_loop(0, ec, lo_body, carry)
            return lax.fori_loop(ec, _IDX_CHUNK, hi_body, acc)

        zero = tuple(jnp.zeros((_LANES,), jnp.float32) for _ in range(vregs))
        gather(0, 0, rows0, sem0)

        def step(s, _):
            gather(s, 1, rows1, sem1)
            cnt = jnp.max(cnt_v[s, pl.ds(0, _LANES)])
            ec0 = jnp.minimum(cnt, _IDX_CHUNK)
            ec1 = jnp.maximum(cnt - _IDX_CHUNK, 0)
            drain(rows0, sem0)
            acc = accum_chunk(rows0, ec0, zero)
            gather(jnp.minimum(s + 1, b_per_w - 1), 0, rows0, sem0)
            drain(rows1, sem1)
            acc = accum_chunk(rows1, ec1, acc)
            for v in range(vregs):
                out_v[s, pl.ds(v * _LANES, _LANES)] = acc[v]
            return 0

        lax.fori_loop(0, b_per_w, step, 0)
        drain(rows0, sem0)
        pltpu.sync_copy(out_v, out_hbm.at[pl.ds(base, b_per_w)])

    return k


def _mlp_body(inv_hist, sums_ref, w1_ref, b1_ref, w2_ref, b2_ref, out_ref):
    pooled = sums_ref[...] * inv_hist
    h = jnp.tanh(
        jnp.dot(pooled, w1_ref[...], preferred_element_type=jnp.float32)
        + b1_ref[...]
    )
    out_ref[...] = (
        jnp.dot(h, w2_ref[...], preferred_element_type=jnp.float32) + b2_ref[...]
    )


def kernel(input_ids, embedding, W1, b1, W2, b2):
    batch, hist = input_ids.shape
    vocab, dim = embedding.shape

    n_chunk = hist // _IDX_CHUNK
    pbit = (vocab - 1).bit_length()
    ids32 = input_ids.astype(jnp.int32)
    par = ids32 & 1
    ids_sorted = jnp.sort(ids32 | (par << pbit), axis=1) & ((1 << pbit) - 1)
    cnt = (hist - jnp.sum(par, axis=1, dtype=jnp.int32)).astype(jnp.int32)
    cnt16 = jnp.broadcast_to(cnt[:, None], (batch, _LANES))
    ids = jnp.pad(
        ids_sorted.reshape(batch, n_chunk, _IDX_CHUNK),
        ((0, 0), (0, 0), (0, _LIST_LEN - _IDX_CHUNK)),
    ).reshape(batch, n_chunk * _LIST_LEN)
    emb2 = embedding.reshape(vocab // 2, 2 * dim)
    sums = _gather_pool_kernel(batch, hist, dim)(ids, cnt16, emb2)

    out = pl.pallas_call(
        functools.partial(_mlp_body, 1.0 / hist),
        out_shape=jax.ShapeDtypeStruct((batch, 1), jnp.float32),
    )(sums, W1, b1.reshape(1, -1), W2, b2.reshape(1, 1))
    return out[:, 0]


# speedup vs baseline: 6.2236x; 6.2137x over previous
import functools

import jax
import jax.numpy as jnp
from jax import lax
from jax.experimental import pallas as pl
from jax.experimental.pallas import tpu as pltpu
from jax.experimental.pallas import tpu_sc as plsc

_NC = 2
_NS = 16
_NW = _NC * _NS
_LANES = 16
_IDX_CHUNK = 100


def _gather_pool_kernel(batch, hist, dim):
    n_chunk = hist // _IDX_CHUNK
    b_per_w = batch // _NW
    vregs = dim // _LANES
    mesh = plsc.VectorSubcoreMesh(core_axis_name="c", subcore_axis_name="s")

    unroll = 4

    @functools.partial(
        pl.kernel,
        mesh=mesh,
        out_type=jax.ShapeDtypeStruct((batch, dim), jnp.float32),
        scratch_types=[
            pltpu.VMEM((b_per_w, n_chunk, _IDX_CHUNK), jnp.int32),
            pltpu.VMEM((hist, dim), jnp.float32),
            pltpu.VMEM((hist, dim), jnp.float32),
            pltpu.VMEM((b_per_w, dim), jnp.float32),
            pltpu.SemaphoreType.DMA,
            pltpu.SemaphoreType.DMA,
        ],
        compiler_params=pltpu.CompilerParams(use_tc_tiling_on_sc=False),
    )
    def k(ids_hbm, emb_hbm, out_hbm, idx_v, rows0, rows1, out_v, sem0, sem1):
        wid = lax.axis_index("s") * _NC + lax.axis_index("c")
        base = wid * b_per_w

        def gather(s, rows, sem):
            for c in range(n_chunk):
                pltpu.async_copy(
                    emb_hbm.at[idx_v.at[s].at[c]],
                    rows.at[pl.ds(c * _IDX_CHUNK, _IDX_CHUNK)],
                    sem,
                )

        def drain(rows, sem):
            for c in range(n_chunk):
                pltpu.make_async_copy(
                    emb_hbm.at[idx_v.at[0].at[c]],
                    rows.at[pl.ds(c * _IDX_CHUNK, _IDX_CHUNK)],
                    sem,
                ).wait()

        def accum(rows, s_out):
            def acc_body(j, carry):
                new = carry
                for u in range(unroll):
                    new = tuple(
                        new[v] + rows[j * unroll + u, pl.ds(v * _LANES, _LANES)]
                        for v in range(vregs)
                    )
                return new

            acc = lax.fori_loop(
                0, hist // unroll, acc_body,
                tuple(jnp.zeros((_LANES,), jnp.float32) for _ in range(vregs)),
            )
            for v in range(vregs):
                out_v[s_out, pl.ds(v * _LANES, _LANES)] = acc[v]

        pltpu.sync_copy(ids_hbm.at[pl.ds(base, b_per_w)], idx_v)
        gather(0, rows0, sem0)

        def step(t, _):
            sa = 2 * t
            sb = 2 * t + 1
            gather(sb, rows1, sem1)
            drain(rows0, sem0)
            accum(rows0, sa)
            gather(jnp.minimum(sa + 2, b_per_w - 1), rows0, sem0)
            drain(rows1, sem1)
            accum(rows1, sb)
            return 0

        lax.fori_loop(0, b_per_w // 2, step, 0)
        drain(rows0, sem0)
        pltpu.sync_copy(out_v, out_hbm.at[pl.ds(base, b_per_w)])

    return k


def _mlp_body(inv_hist, sums_ref, w1_ref, b1_ref, w2_ref, b2_ref, out_ref):
    pooled = sums_ref[...] * inv_hist
    h = jnp.tanh(
        jnp.dot(pooled, w1_ref[...], preferred_element_type=jnp.float32)
        + b1_ref[...]
    )
    out_ref[...] = (
        jnp.dot(h, w2_ref[...], preferred_element_type=jnp.float32) + b2_ref[...]
    )


def kernel(input_ids, embedding, W1, b1, W2, b2):
    batch, hist = input_ids.shape
    _, dim = embedding.shape
    n_chunk = hist // _IDX_CHUNK

    ids = input_ids.astype(jnp.int32).reshape(batch, n_chunk, _IDX_CHUNK)
    sums = _gather_pool_kernel(batch, hist, dim)(ids, embedding)

    out = pl.pallas_call(
        functools.partial(_mlp_body, 1.0 / hist),
        out_shape=jax.ShapeDtypeStruct((batch, 1), jnp.float32),
    )(sums, W1, b1.reshape(1, -1), W2, b2.reshape(1, 1))
    return out[:, 0]


# speedup vs baseline: 6.2306x; 1.0011x over previous
import functools

import jax
import jax.numpy as jnp
from jax import lax
from jax.experimental import pallas as pl
from jax.experimental.pallas import tpu as pltpu
from jax.experimental.pallas import tpu_sc as plsc

_NC = 2
_NS = 16
_NW = _NC * _NS
_LANES = 16
_IDX_CHUNK = 100


def _gather_pool_kernel(batch, hist, dim):
    n_chunk = hist // _IDX_CHUNK
    b_per_w = batch // _NW
    vregs = dim // _LANES
    mesh = plsc.VectorSubcoreMesh(core_axis_name="c", subcore_axis_name="s")

    unroll = 8

    @functools.partial(
        pl.kernel,
        mesh=mesh,
        out_type=jax.ShapeDtypeStruct((batch, dim), jnp.float32),
        scratch_types=[
            pltpu.VMEM((b_per_w, n_chunk, _IDX_CHUNK), jnp.int32),
            pltpu.VMEM((hist, dim), jnp.float32),
            pltpu.VMEM((hist, dim), jnp.float32),
            pltpu.VMEM((b_per_w, dim), jnp.float32),
            pltpu.SemaphoreType.DMA,
            pltpu.SemaphoreType.DMA,
        ],
        compiler_params=pltpu.CompilerParams(use_tc_tiling_on_sc=False),
    )
    def k(ids_hbm, emb_hbm, out_hbm, idx_v, rows0, rows1, out_v, sem0, sem1):
        wid = lax.axis_index("s") * _NC + lax.axis_index("c")
        base = wid * b_per_w

        def gather(s, rows, sem):
            for c in range(n_chunk):
                pltpu.async_copy(
                    emb_hbm.at[idx_v.at[s].at[c]],
                    rows.at[pl.ds(c * _IDX_CHUNK, _IDX_CHUNK)],
                    sem,
                )

        def drain(rows, sem):
            for c in range(n_chunk):
                pltpu.make_async_copy(
                    emb_hbm.at[idx_v.at[0].at[c]],
                    rows.at[pl.ds(c * _IDX_CHUNK, _IDX_CHUNK)],
                    sem,
                ).wait()

        def accum(rows, s_out):
            def acc_body(j, carry):
                new = carry
                for u in range(unroll):
                    new = tuple(
                        new[v] + rows[j * unroll + u, pl.ds(v * _LANES, _LANES)]
                        for v in range(vregs)
                    )
                return new

            acc = lax.fori_loop(
                0, hist // unroll, acc_body,
                tuple(jnp.zeros((_LANES,), jnp.float32) for _ in range(vregs)),
            )
            for v in range(vregs):
                out_v[s_out, pl.ds(v * _LANES, _LANES)] = acc[v]

        pltpu.sync_copy(ids_hbm.at[pl.ds(base, b_per_w)], idx_v)
        gather(0, rows0, sem0)

        def step(t, _):
            sa = 2 * t
            sb = 2 * t + 1
            gather(sb, rows1, sem1)
            drain(rows0, sem0)
            accum(rows0, sa)
            gather(jnp.minimum(sa + 2, b_per_w - 1), rows0, sem0)
            drain(rows1, sem1)
            accum(rows1, sb)
            return 0

        lax.fori_loop(0, b_per_w // 2, step, 0)
        drain(rows0, sem0)
        pltpu.sync_copy(out_v, out_hbm.at[pl.ds(base, b_per_w)])

    return k


def _mlp_body(inv_hist, sums_ref, w1_ref, b1_ref, w2_ref, b2_ref, out_ref):
    pooled = sums_ref[...] * inv_hist
    h = jnp.tanh(
        jnp.dot(pooled, w1_ref[...], preferred_element_type=jnp.float32)
        + b1_ref[...]
    )
    out_ref[...] = (
        jnp.dot(h, w2_ref[...], preferred_element_type=jnp.float32) + b2_ref[...]
    )


def kernel(input_ids, embedding, W1, b1, W2, b2):
    batch, hist = input_ids.shape
    _, dim = embedding.shape
    n_chunk = hist // _IDX_CHUNK

    ids = input_ids.astype(jnp.int32).reshape(batch, n_chunk, _IDX_CHUNK)
    sums = _gather_pool_kernel(batch, hist, dim)(ids, embedding)

    out = pl.pallas_call(
        functools.partial(_mlp_body, 1.0 / hist),
        out_shape=jax.ShapeDtypeStruct((batch, 1), jnp.float32),
    )(sums, W1, b1.reshape(1, -1), W2, b2.reshape(1, 1))
    return out[:, 0]
